# baseline (speedup 1.0000x reference)
import dataclasses

import jax
import jax.numpy as jnp
from jax import lax
from jax.experimental import pallas as pl
from jax.experimental.pallas import tpu as pltpu
from jax.experimental.pallas import tpu_sc as plsc

NCLS = 80
RMAX = 16
BOX_GAIN, CLS_GAIN, DFL_GAIN = 7.5, 0.5, 1.5
STRIDES = (8.0, 16.0, 32.0)
EPS = 1e-07
B = 32
N = 400
SHAPES = ((64, 64), (32, 32), (16, 16))

_INTERPRET = False


def _dense_body(p0, p1, p2, o0, o1, o2):
    i = pl.program_id(0)
    for ref, o in ((p0, o0), (p1, o1), (p2, o2)):

        @pl.when(i == 0)
        def _():
            o[...] = jnp.zeros_like(o)

        def step(c, carry):
            x = ref[0, c + 4]
            f = jnp.maximum(x, 0.0) + jnp.log1p(jnp.exp(-jnp.abs(x)))
            o[c] += f
            return carry

        lax.fori_loop(0, 80, step, 0)


def _iou(px, py, pw, ph, tx, ty, tw, th):
    b1x1 = px - pw / 2
    b1x2 = px + pw / 2
    b1y1 = py - ph / 2
    b1y2 = py + ph / 2
    b2x1 = tx - tw / 2
    b2x2 = tx + tw / 2
    b2y1 = ty - th / 2
    b2y2 = ty + th / 2
    inter = (jnp.clip(jnp.minimum(b1x2, b2x2) - jnp.maximum(b1x1, b2x1), 0, None)
             * jnp.clip(jnp.minimum(b1y2, b2y2) - jnp.maximum(b1y1, b2y1), 0, None))
    w1, h1 = b1x2 - b1x1, b1y2 - b1y1 + EPS
    w2, h2 = b2x2 - b2x1, b2y2 - b2y1 + EPS
    union = w1 * h1 + w2 * h2 - inter + EPS
    return inter / union


def _combine_body(tt2, tt3, gp0, gp1, gp2, xp0, xp1, xp2, gd0, gd1, gd2,
                  d0, d1, d2, o):
    bi2 = tt2[0:1, :].astype(jnp.int32)
    ci2 = tt2[1:2, :].astype(jnp.int32)
    x2t = tt2[2:3, :]
    y2t = tt2[3:4, :]
    x3 = tt3[2]
    y3 = tt3[3]
    w3 = tt3[4]
    h3 = tt3[5]
    loss_box = jnp.float32(0.0)
    loss_cls = jnp.float32(0.0)
    loss_dfl = jnp.float32(0.0)
    for s, (gp, xp, gd, dd) in enumerate(((gp0, xp0, gd0, d0),
                                          (gp1, xp1, gd1, d1),
                                          (gp2, xp2, gd2, d2))):
        H, W = SHAPES[s]
        stride = STRIDES[s]
        sw = jnp.float32(W / stride)
        sh = jnp.float32(H / stride)
        g0 = x3 * sw
        g1 = y3 * sh
        gif = jnp.floor(g0)
        gjf = jnp.floor(g1)
        tbx = g0 - gif
        tby = g1 - gjf
        tbw = w3 * sw
        tbh = h3 * sh
        iou = _iou(gp[0], gp[1], gp[2], gp[3], tbx, tby, tbw, tbh)
        loss_box = loss_box + jnp.sum(1.0 - iou) * jnp.float32(1.0 / N)
        gi2 = jnp.floor(x2t * sw).astype(jnp.int32)
        gj2 = jnp.floor(y2t * sh).astype(jnp.int32)
        flat = bi2 * (H * W) + gj2 * W + gi2
        key = flat * NCLS + ci2
        keyc = jnp.transpose(key)
        eq = (keyc == key)
        earlier = (jax.lax.broadcasted_iota(jnp.int32, (N, N), 1)
                   < jax.lax.broadcasted_iota(jnp.int32, (N, N), 0))
        dup = jnp.sum((eq & earlier).astype(jnp.int32), axis=1, keepdims=True)
        keep = jnp.transpose((dup == 0).astype(jnp.float32))
        possum = jnp.sum(xp[...] * keep)
        ssum = jnp.sum(dd[...])
        loss_cls = loss_cls + (ssum - possum) * jnp.float32(1.0 / (B * H * W * NCLS))
        tbxs = tbx * W
        tbys = tby * H
        tbws = tbw * W
        tbhs = tbh * H
        cx1 = tbxs - tbws / 2
        cy1 = tbys - tbhs / 2
        cx2 = tbxs + tbws / 2
        cy2 = tbys + tbhs / 2
        for j, corner in enumerate((cx1, cy1, cx2, cy2)):
            ccl = jnp.clip(corner, 0.0, float(RMAX - 1))
            tgt = jnp.clip(jnp.round(ccl), 0.0, float(RMAX - 1)).astype(jnp.int32)
            logits = gd[:, 16 * j:16 * j + 16, :]
            m = jnp.max(logits, axis=1, keepdims=True)
            se = jnp.sum(jnp.exp(logits - m), axis=1, keepdims=True)
            lse = jnp.log(se) + m
            krow = jax.lax.broadcasted_iota(jnp.int32, (NG, RMAX, 16), 1)
            lt = jnp.sum(jnp.where(krow == tgt[:, None, :], logits, 0.0),
                         axis=1, keepdims=True)
            loss_dfl = loss_dfl + jnp.sum(lse - lt)
    loss_dfl = loss_dfl * jnp.float32(1.0 / (N * 4))
    lb = loss_box * BOX_GAIN
    lc = loss_cls * CLS_GAIN
    ld = loss_dfl * DFL_GAIN
    tot = lb + lc + ld
    lane = jax.lax.broadcasted_iota(jnp.int32, (1, 4), 1)
    o[...] = jnp.where(lane == 0, tot,
                       jnp.where(lane == 1, lb, jnp.where(lane == 2, lc, ld)))


NG = N // 16
GJMAX = (8, 2, 1)


def _sc_gather_body(tt, t0, t1, t2,
                    gp0, gp1, gp2, xp0, xp1, xp2, gd0, gd1, gd2,
                    tv, ix0, ix1, ix2, rd0, rd1, rd2,
                    outp, outd, sem):
    wid = lax.axis_index("c") * 16 + lax.axis_index("s")

    @pl.when(wid < NG)
    def _():
        g16 = wid * 16
        for j in range(6):
            pltpu.sync_copy(tt.at[j, pl.ds(g16, 16)], tv.at[j])
        bi = tv[0].astype(jnp.int32)
        ci = tv[1].astype(jnp.int32)
        xv = tv[2]
        yv = tv[3]
        iota16 = lax.iota(jnp.int32, 16)
        tabs = (t0, t1, t2)
        ixrefs = (ix0, ix1, ix2)
        rdrefs = (rd0, rd1, rd2)
        gprefs = (gp0, gp1, gp2)
        xprefs = (xp0, xp1, xp2)
        gdrefs = (gd0, gd1, gd2)
        copies = []
        for s in range(3):
            H, W = SHAPES[s]
            stride = STRIDES[s]
            g0 = xv * jnp.float32(W / stride)
            g1 = yv * jnp.float32(H / stride)
            gi = g0.astype(jnp.int32)
            gj = g1.astype(jnp.int32)
            gm = GJMAX[s]
            ixrefs[s][...] = (bi * gm + gj) * gm + gi
            copies.append(pltpu.async_copy(
                tabs[s].at[ixrefs[s]], rdrefs[s], sem))
        for cp in copies:
            cp.wait()
        for s in range(3):
            rd = rdrefs[s]
            for c in range(4):
                outp[c] = plsc.load_gather(
                    rd, [iota16, jnp.full((16,), 64 + c, jnp.int32)])
            outp[4] = plsc.load_gather(rd, [iota16, 68 + ci])
            for c in range(5):
                pltpu.sync_copy(outp.at[c],
                                gprefs[s].at[pl.ds(c * N + g16, 16)])
            pltpu.sync_copy(outp.at[4], xprefs[s].at[pl.ds(g16, 16)])
            for c in range(64):
                outd[pl.ds(c * 16, 16)] = plsc.load_gather(
                    rd, [iota16, jnp.full((16,), c, jnp.int32)])
            pltpu.sync_copy(outd, gdrefs[s].at[pl.ds(wid * 1024, 1024)])


def _sc_gather(tt, t0, t1, t2):
    f32 = jnp.float32
    cp = pltpu.CompilerParams()
    fields = pltpu.CompilerParams.__dataclass_fields__
    if "needs_layout_passes" in fields:
        cp = dataclasses.replace(cp, needs_layout_passes=False)
    return pl.kernel(
        _sc_gather_body,
        compiler_params=cp,
        out_type=(
            jax.ShapeDtypeStruct((5 * N,), f32),
            jax.ShapeDtypeStruct((5 * N,), f32),
            jax.ShapeDtypeStruct((5 * N,), f32),
            jax.ShapeDtypeStruct((N,), f32),
            jax.ShapeDtypeStruct((N,), f32),
            jax.ShapeDtypeStruct((N,), f32),
            jax.ShapeDtypeStruct((64 * N,), f32),
            jax.ShapeDtypeStruct((64 * N,), f32),
            jax.ShapeDtypeStruct((64 * N,), f32),
        ),
        mesh=plsc.VectorSubcoreMesh(core_axis_name="c", subcore_axis_name="s"),
        scratch_types=[
            pltpu.VMEM((8, 16), f32),
            pltpu.VMEM((16,), jnp.int32),
            pltpu.VMEM((16,), jnp.int32),
            pltpu.VMEM((16,), jnp.int32),
            pltpu.VMEM((16, 256), f32),
            pltpu.VMEM((16, 256), f32),
            pltpu.VMEM((16, 256), f32),
            pltpu.VMEM((8, 16), f32),
            pltpu.VMEM((1024,), f32),
            pltpu.SemaphoreType.DMA,
        ],
    )(tt, t0, t1, t2)


def _prep_body(p0, d0, p1, d1, p2, d2, t0, t1, t2):
    for s, (p, d, t) in enumerate(((p0, d0, t0), (p1, d1, t1), (p2, d2, t2))):
        gm = GJMAX[s]
        cells = gm * gm
        pt = jnp.transpose(p[0, :, 0:gm, 0:gm], (1, 2, 0)).reshape(cells, 84)
        dt = jnp.transpose(d[0, :, 0:gm, 0:gm], (1, 2, 0)).reshape(cells, 64)
        t[0] = jnp.concatenate(
            [dt, pt, jnp.zeros((cells, 108), jnp.float32)], axis=1)


def _slab_tables(pred0, dfl0, pred1, dfl1, pred2, dfl2):
    return pl.pallas_call(
        _prep_body,
        grid=(B,),
        in_specs=[
            pl.BlockSpec((1, 84, 8, 64), lambda b: (b, 0, 0, 0)),
            pl.BlockSpec((1, 64, 8, 64), lambda b: (b, 0, 0, 0)),
            pl.BlockSpec((1, 84, 8, 32), lambda b: (b, 0, 0, 0)),
            pl.BlockSpec((1, 64, 8, 32), lambda b: (b, 0, 0, 0)),
            pl.BlockSpec((1, 84, 8, 16), lambda b: (b, 0, 0, 0)),
            pl.BlockSpec((1, 64, 8, 16), lambda b: (b, 0, 0, 0)),
        ],
        out_specs=[
            pl.BlockSpec((1, 64, 256), lambda b: (b, 0, 0)),
            pl.BlockSpec((1, 4, 256), lambda b: (b, 0, 0)),
            pl.BlockSpec((1, 1, 256), lambda b: (b, 0, 0)),
        ],
        out_shape=[
            jax.ShapeDtypeStruct((B, 64, 256), jnp.float32),
            jax.ShapeDtypeStruct((B, 4, 256), jnp.float32),
            jax.ShapeDtypeStruct((B, 1, 256), jnp.float32),
        ],
        interpret=_INTERPRET,
    )(pred0, dfl0, pred1, dfl1, pred2, dfl2)


def kernel(pred0, pred1, pred2, dfl0, dfl1, dfl2, targets):
    tt = targets.T
    t0, t1, t2 = _slab_tables(pred0, dfl0, pred1, dfl1, pred2, dfl2)
    gp0, gp1, gp2, xp0, xp1, xp2, gd0, gd1, gd2 = _sc_gather(
        tt, t0.reshape(B * 64, 256), t1.reshape(B * 4, 256),
        t2.reshape(B, 256))
    gp0, gp1, gp2 = (g.reshape(5, NG, 16) for g in (gp0, gp1, gp2))
    xp0, xp1, xp2 = (xg.reshape(1, N) for xg in (xp0, xp1, xp2))
    gd0, gd1, gd2 = (g.reshape(NG, 64, 16) for g in (gd0, gd1, gd2))
    tt3 = tt.reshape(6, NG, 16)

    d0, d1, d2 = pl.pallas_call(
        _dense_body,
        grid=(B,),
        in_specs=[
            pl.BlockSpec((1, 84, 64, 64), lambda b: (b, 0, 0, 0)),
            pl.BlockSpec((1, 84, 32, 32), lambda b: (b, 0, 0, 0)),
            pl.BlockSpec((1, 84, 16, 16), lambda b: (b, 0, 0, 0)),
        ],
        out_specs=[
            pl.BlockSpec((80, 64, 64), lambda b: (0, 0, 0)),
            pl.BlockSpec((80, 32, 32), lambda b: (0, 0, 0)),
            pl.BlockSpec((80, 16, 16), lambda b: (0, 0, 0)),
        ],
        out_shape=[
            jax.ShapeDtypeStruct((80, 64, 64), jnp.float32),
            jax.ShapeDtypeStruct((80, 32, 32), jnp.float32),
            jax.ShapeDtypeStruct((80, 16, 16), jnp.float32),
        ],
        interpret=_INTERPRET,
    )(pred0, pred1, pred2)

    out = pl.pallas_call(
        _combine_body,
        out_shape=jax.ShapeDtypeStruct((1, 4), jnp.float32),
        interpret=_INTERPRET,
    )(tt, tt3, gp0, gp1, gp2, xp0, xp1, xp2, gd0, gd1, gd2, d0, d1, d2)
    return out.reshape(4)


# speedup vs baseline: 1.3618x; 1.3618x over previous
import dataclasses

import jax
import jax.numpy as jnp
from jax import lax
from jax.experimental import pallas as pl
from jax.experimental.pallas import tpu as pltpu
from jax.experimental.pallas import tpu_sc as plsc

NCLS = 80
RMAX = 16
BOX_GAIN, CLS_GAIN, DFL_GAIN = 7.5, 0.5, 1.5
STRIDES = (8.0, 16.0, 32.0)
EPS = 1e-07
B = 32
N = 400
SHAPES = ((64, 64), (32, 32), (16, 16))

_INTERPRET = False


def _dense_body(p0, p1, p2, o0, o1, o2):
    i = pl.program_id(0)
    for ref, o in ((p0, o0), (p1, o1), (p2, o2)):

        @pl.when(i == 0)
        def _():
            o[...] = jnp.zeros_like(o)

        for c in range(0, 80, 2):
            x = ref[0, c + 4:c + 6]
            ax = jnp.abs(x)
            f = 0.5 * (x + ax) + jnp.log(1.0 + jnp.exp(-ax))
            o[c:c + 2] += f


def _iou(px, py, pw, ph, tx, ty, tw, th):
    b1x1 = px - pw / 2
    b1x2 = px + pw / 2
    b1y1 = py - ph / 2
    b1y2 = py + ph / 2
    b2x1 = tx - tw / 2
    b2x2 = tx + tw / 2
    b2y1 = ty - th / 2
    b2y2 = ty + th / 2
    inter = (jnp.clip(jnp.minimum(b1x2, b2x2) - jnp.maximum(b1x1, b2x1), 0, None)
             * jnp.clip(jnp.minimum(b1y2, b2y2) - jnp.maximum(b1y1, b2y1), 0, None))
    w1, h1 = b1x2 - b1x1, b1y2 - b1y1 + EPS
    w2, h2 = b2x2 - b2x1, b2y2 - b2y1 + EPS
    union = w1 * h1 + w2 * h2 - inter + EPS
    return inter / union


def _combine_body(tt2, tt3, gp0, gp1, gp2, xp0, xp1, xp2, gd0, gd1, gd2,
                  d0, d1, d2, o):
    bi2 = tt2[0:1, :].astype(jnp.int32)
    ci2 = tt2[1:2, :].astype(jnp.int32)
    x2t = tt2[2:3, :]
    y2t = tt2[3:4, :]
    x3 = tt3[2]
    y3 = tt3[3]
    w3 = tt3[4]
    h3 = tt3[5]
    loss_box = jnp.float32(0.0)
    loss_cls = jnp.float32(0.0)
    loss_dfl = jnp.float32(0.0)
    for s, (gp, xp, gd, dd) in enumerate(((gp0, xp0, gd0, d0),
                                          (gp1, xp1, gd1, d1),
                                          (gp2, xp2, gd2, d2))):
        H, W = SHAPES[s]
        stride = STRIDES[s]
        sw = jnp.float32(W / stride)
        sh = jnp.float32(H / stride)
        g0 = x3 * sw
        g1 = y3 * sh
        gif = jnp.floor(g0)
        gjf = jnp.floor(g1)
        tbx = g0 - gif
        tby = g1 - gjf
        tbw = w3 * sw
        tbh = h3 * sh
        iou = _iou(gp[0], gp[1], gp[2], gp[3], tbx, tby, tbw, tbh)
        loss_box = loss_box + jnp.sum(1.0 - iou) * jnp.float32(1.0 / N)
        gi2 = jnp.floor(x2t * sw).astype(jnp.int32)
        gj2 = jnp.floor(y2t * sh).astype(jnp.int32)
        flat = bi2 * (H * W) + gj2 * W + gi2
        key = flat * NCLS + ci2
        keyc = jnp.transpose(key)
        eq = (keyc == key)
        earlier = (jax.lax.broadcasted_iota(jnp.int32, (N, N), 1)
                   < jax.lax.broadcasted_iota(jnp.int32, (N, N), 0))
        dup = jnp.sum((eq & earlier).astype(jnp.int32), axis=1, keepdims=True)
        keep = jnp.transpose((dup == 0).astype(jnp.float32))
        possum = jnp.sum(xp[...] * keep)
        ssum = jnp.sum(dd[...])
        loss_cls = loss_cls + (ssum - possum) * jnp.float32(1.0 / (B * H * W * NCLS))
        tbxs = tbx * W
        tbys = tby * H
        tbws = tbw * W
        tbhs = tbh * H
        cx1 = tbxs - tbws / 2
        cy1 = tbys - tbhs / 2
        cx2 = tbxs + tbws / 2
        cy2 = tbys + tbhs / 2
        for j, corner in enumerate((cx1, cy1, cx2, cy2)):
            ccl = jnp.clip(corner, 0.0, float(RMAX - 1))
            tgt = jnp.clip(jnp.round(ccl), 0.0, float(RMAX - 1)).astype(jnp.int32)
            logits = gd[:, 16 * j:16 * j + 16, :]
            m = jnp.max(logits, axis=1, keepdims=True)
            se = jnp.sum(jnp.exp(logits - m), axis=1, keepdims=True)
            lse = jnp.log(se) + m
            krow = jax.lax.broadcasted_iota(jnp.int32, (NG, RMAX, 16), 1)
            lt = jnp.sum(jnp.where(krow == tgt[:, None, :], logits, 0.0),
                         axis=1, keepdims=True)
            loss_dfl = loss_dfl + jnp.sum(lse - lt)
    loss_dfl = loss_dfl * jnp.float32(1.0 / (N * 4))
    lb = loss_box * BOX_GAIN
    lc = loss_cls * CLS_GAIN
    ld = loss_dfl * DFL_GAIN
    tot = lb + lc + ld
    lane = jax.lax.broadcasted_iota(jnp.int32, (1, 4), 1)
    o[...] = jnp.where(lane == 0, tot,
                       jnp.where(lane == 1, lb, jnp.where(lane == 2, lc, ld)))


NG = N // 16
GJMAX = (8, 2, 1)


def _sc_gather_body(tt, t0, t1, t2,
                    gp0, gp1, gp2, xp0, xp1, xp2, gd0, gd1, gd2,
                    tv, ix0, ix1, ix2, rd0, rd1, rd2,
                    outp, outd, sem):
    wid = lax.axis_index("c") * 16 + lax.axis_index("s")

    @pl.when(wid < NG)
    def _():
        g16 = wid * 16
        for j in range(6):
            pltpu.sync_copy(tt.at[j, pl.ds(g16, 16)], tv.at[j])
        bi = tv[0].astype(jnp.int32)
        ci = tv[1].astype(jnp.int32)
        xv = tv[2]
        yv = tv[3]
        iota16 = lax.iota(jnp.int32, 16)
        tabs = (t0, t1, t2)
        ixrefs = (ix0, ix1, ix2)
        rdrefs = (rd0, rd1, rd2)
        gprefs = (gp0, gp1, gp2)
        xprefs = (xp0, xp1, xp2)
        gdrefs = (gd0, gd1, gd2)
        copies = []
        for s in range(3):
            H, W = SHAPES[s]
            stride = STRIDES[s]
            g0 = xv * jnp.float32(W / stride)
            g1 = yv * jnp.float32(H / stride)
            gi = g0.astype(jnp.int32)
            gj = g1.astype(jnp.int32)
            gm = GJMAX[s]
            ixrefs[s][...] = (bi * gm + gj) * gm + gi
            copies.append(pltpu.async_copy(
                tabs[s].at[ixrefs[s]], rdrefs[s], sem))
        for cp in copies:
            cp.wait()
        for s in range(3):
            rd = rdrefs[s]
            for c in range(4):
                outp[c] = plsc.load_gather(
                    rd, [iota16, jnp.full((16,), 64 + c, jnp.int32)])
            outp[4] = plsc.load_gather(rd, [iota16, 68 + ci])
            for c in range(5):
                pltpu.sync_copy(outp.at[c],
                                gprefs[s].at[pl.ds(c * N + g16, 16)])
            pltpu.sync_copy(outp.at[4], xprefs[s].at[pl.ds(g16, 16)])
            for c in range(64):
                outd[pl.ds(c * 16, 16)] = plsc.load_gather(
                    rd, [iota16, jnp.full((16,), c, jnp.int32)])
            pltpu.sync_copy(outd, gdrefs[s].at[pl.ds(wid * 1024, 1024)])


def _sc_gather(tt, t0, t1, t2):
    f32 = jnp.float32
    cp = pltpu.CompilerParams()
    fields = pltpu.CompilerParams.__dataclass_fields__
    if "needs_layout_passes" in fields:
        cp = dataclasses.replace(cp, needs_layout_passes=False)
    return pl.kernel(
        _sc_gather_body,
        compiler_params=cp,
        out_type=(
            jax.ShapeDtypeStruct((5 * N,), f32),
            jax.ShapeDtypeStruct((5 * N,), f32),
            jax.ShapeDtypeStruct((5 * N,), f32),
            jax.ShapeDtypeStruct((N,), f32),
            jax.ShapeDtypeStruct((N,), f32),
            jax.ShapeDtypeStruct((N,), f32),
            jax.ShapeDtypeStruct((64 * N,), f32),
            jax.ShapeDtypeStruct((64 * N,), f32),
            jax.ShapeDtypeStruct((64 * N,), f32),
        ),
        mesh=plsc.VectorSubcoreMesh(core_axis_name="c", subcore_axis_name="s"),
        scratch_types=[
            pltpu.VMEM((8, 16), f32),
            pltpu.VMEM((16,), jnp.int32),
            pltpu.VMEM((16,), jnp.int32),
            pltpu.VMEM((16,), jnp.int32),
            pltpu.VMEM((16, 256), f32),
            pltpu.VMEM((16, 256), f32),
            pltpu.VMEM((16, 256), f32),
            pltpu.VMEM((8, 16), f32),
            pltpu.VMEM((1024,), f32),
            pltpu.SemaphoreType.DMA,
        ],
    )(tt, t0, t1, t2)


def _prep_body(p0, d0, p1, d1, p2, d2, t0, t1, t2):
    for s, (p, d, t) in enumerate(((p0, d0, t0), (p1, d1, t1), (p2, d2, t2))):
        gm = GJMAX[s]
        cells = gm * gm
        pt = jnp.transpose(p[0, :, 0:gm, 0:gm], (1, 2, 0)).reshape(cells, 84)
        dt = jnp.transpose(d[0, :, 0:gm, 0:gm], (1, 2, 0)).reshape(cells, 64)
        t[0] = jnp.concatenate(
            [dt, pt, jnp.zeros((cells, 108), jnp.float32)], axis=1)


def _slab_tables(pred0, dfl0, pred1, dfl1, pred2, dfl2):
    return pl.pallas_call(
        _prep_body,
        grid=(B,),
        in_specs=[
            pl.BlockSpec((1, 84, 8, 64), lambda b: (b, 0, 0, 0)),
            pl.BlockSpec((1, 64, 8, 64), lambda b: (b, 0, 0, 0)),
            pl.BlockSpec((1, 84, 8, 32), lambda b: (b, 0, 0, 0)),
            pl.BlockSpec((1, 64, 8, 32), lambda b: (b, 0, 0, 0)),
            pl.BlockSpec((1, 84, 8, 16), lambda b: (b, 0, 0, 0)),
            pl.BlockSpec((1, 64, 8, 16), lambda b: (b, 0, 0, 0)),
        ],
        out_specs=[
            pl.BlockSpec((1, 64, 256), lambda b: (b, 0, 0)),
            pl.BlockSpec((1, 4, 256), lambda b: (b, 0, 0)),
            pl.BlockSpec((1, 1, 256), lambda b: (b, 0, 0)),
        ],
        out_shape=[
            jax.ShapeDtypeStruct((B, 64, 256), jnp.float32),
            jax.ShapeDtypeStruct((B, 4, 256), jnp.float32),
            jax.ShapeDtypeStruct((B, 1, 256), jnp.float32),
        ],
        interpret=_INTERPRET,
    )(pred0, dfl0, pred1, dfl1, pred2, dfl2)


def kernel(pred0, pred1, pred2, dfl0, dfl1, dfl2, targets):
    tt = targets.T
    t0, t1, t2 = _slab_tables(pred0, dfl0, pred1, dfl1, pred2, dfl2)
    gp0, gp1, gp2, xp0, xp1, xp2, gd0, gd1, gd2 = _sc_gather(
        tt, t0.reshape(B * 64, 256), t1.reshape(B * 4, 256),
        t2.reshape(B, 256))
    gp0, gp1, gp2 = (g.reshape(5, NG, 16) for g in (gp0, gp1, gp2))
    xp0, xp1, xp2 = (xg.reshape(1, N) for xg in (xp0, xp1, xp2))
    gd0, gd1, gd2 = (g.reshape(NG, 64, 16) for g in (gd0, gd1, gd2))
    tt3 = tt.reshape(6, NG, 16)

    d0, d1, d2 = pl.pallas_call(
        _dense_body,
        grid=(B,),
        in_specs=[
            pl.BlockSpec((1, 84, 64, 64), lambda b: (b, 0, 0, 0)),
            pl.BlockSpec((1, 84, 32, 32), lambda b: (b, 0, 0, 0)),
            pl.BlockSpec((1, 84, 16, 16), lambda b: (b, 0, 0, 0)),
        ],
        out_specs=[
            pl.BlockSpec((80, 64, 64), lambda b: (0, 0, 0)),
            pl.BlockSpec((80, 32, 32), lambda b: (0, 0, 0)),
            pl.BlockSpec((80, 16, 16), lambda b: (0, 0, 0)),
        ],
        out_shape=[
            jax.ShapeDtypeStruct((80, 64, 64), jnp.float32),
            jax.ShapeDtypeStruct((80, 32, 32), jnp.float32),
            jax.ShapeDtypeStruct((80, 16, 16), jnp.float32),
        ],
        interpret=_INTERPRET,
    )(pred0, pred1, pred2)

    out = pl.pallas_call(
        _combine_body,
        out_shape=jax.ShapeDtypeStruct((1, 4), jnp.float32),
        interpret=_INTERPRET,
    )(tt, tt3, gp0, gp1, gp2, xp0, xp1, xp2, gd0, gd1, gd2, d0, d1, d2)
    return out.reshape(4)


# speedup vs baseline: 4.1949x; 3.0805x over previous
import dataclasses

import jax
import jax.numpy as jnp
from jax import lax
from jax.experimental import pallas as pl
from jax.experimental.pallas import tpu as pltpu
from jax.experimental.pallas import tpu_sc as plsc

NCLS = 80
RMAX = 16
BOX_GAIN, CLS_GAIN, DFL_GAIN = 7.5, 0.5, 1.5
STRIDES = (8.0, 16.0, 32.0)
EPS = 1e-07
B = 32
N = 400
SHAPES = ((64, 64), (32, 32), (16, 16))

_INTERPRET = False


def _dense_body(p0, p1, p2, d0, d1, d2, o0, o1, o2, t0, t1, t2):
    i = pl.program_id(0)
    for s, (ref, o) in enumerate(((p0, o0), (p1, o1), (p2, o2))):
        H, W = SHAPES[s]

        @pl.when(i == 0)
        def _():
            o[...] = jnp.zeros_like(o)

        for h in range(0, H, 4):
            x = ref[0, h:h + 4]
            ax = jnp.abs(x)
            f = 0.5 * (x + ax) + jnp.log(1.0 + jnp.exp(-ax))
            o[h:h + 4] += f

    z0 = jnp.zeros((64, 108), jnp.float32)
    dt0 = jnp.transpose(d0[0, :, 0:8, 0:8], (1, 2, 0)).reshape(64, 64)
    t0[0] = jnp.concatenate([dt0, p0[0, 0:8, 0:8, :].reshape(64, 84), z0],
                            axis=1)
    t1[0] = jnp.concatenate(
        [d1[0, 0:2, 0:2, :].reshape(4, 64), p1[0, 0:2, 0:2, :].reshape(4, 84),
         jnp.zeros((4, 108), jnp.float32)], axis=1)
    t2[0] = jnp.concatenate(
        [d2[0, 0:1, 0:1, :].reshape(1, 64), p2[0, 0:1, 0:1, :].reshape(1, 84),
         jnp.zeros((1, 108), jnp.float32)], axis=1)


def _iou(px, py, pw, ph, tx, ty, tw, th):
    b1x1 = px - pw / 2
    b1x2 = px + pw / 2
    b1y1 = py - ph / 2
    b1y2 = py + ph / 2
    b2x1 = tx - tw / 2
    b2x2 = tx + tw / 2
    b2y1 = ty - th / 2
    b2y2 = ty + th / 2
    inter = (jnp.clip(jnp.minimum(b1x2, b2x2) - jnp.maximum(b1x1, b2x1), 0, None)
             * jnp.clip(jnp.minimum(b1y2, b2y2) - jnp.maximum(b1y1, b2y1), 0, None))
    w1, h1 = b1x2 - b1x1, b1y2 - b1y1 + EPS
    w2, h2 = b2x2 - b2x1, b2y2 - b2y1 + EPS
    union = w1 * h1 + w2 * h2 - inter + EPS
    return inter / union


def _combine_body(tt2, tt3, gp0, gp1, gp2, xp0, xp1, xp2, gd0, gd1, gd2,
                  d0, d1, d2, o):
    bi2 = tt2[0:1, :].astype(jnp.int32)
    ci2 = tt2[1:2, :].astype(jnp.int32)
    x2t = tt2[2:3, :]
    y2t = tt2[3:4, :]
    x3 = tt3[2]
    y3 = tt3[3]
    w3 = tt3[4]
    h3 = tt3[5]
    loss_box = jnp.float32(0.0)
    loss_cls = jnp.float32(0.0)
    loss_dfl = jnp.float32(0.0)
    for s, (gp, xp, gd, dd) in enumerate(((gp0, xp0, gd0, d0),
                                          (gp1, xp1, gd1, d1),
                                          (gp2, xp2, gd2, d2))):
        H, W = SHAPES[s]
        stride = STRIDES[s]
        sw = jnp.float32(W / stride)
        sh = jnp.float32(H / stride)
        g0 = x3 * sw
        g1 = y3 * sh
        gif = jnp.floor(g0)
        gjf = jnp.floor(g1)
        tbx = g0 - gif
        tby = g1 - gjf
        tbw = w3 * sw
        tbh = h3 * sh
        iou = _iou(gp[0], gp[1], gp[2], gp[3], tbx, tby, tbw, tbh)
        loss_box = loss_box + jnp.sum(1.0 - iou) * jnp.float32(1.0 / N)
        gi2 = jnp.floor(x2t * sw).astype(jnp.int32)
        gj2 = jnp.floor(y2t * sh).astype(jnp.int32)
        flat = bi2 * (H * W) + gj2 * W + gi2
        key = flat * NCLS + ci2
        keyc = jnp.transpose(key)
        eq = (keyc == key)
        earlier = (jax.lax.broadcasted_iota(jnp.int32, (N, N), 1)
                   < jax.lax.broadcasted_iota(jnp.int32, (N, N), 0))
        dup = jnp.sum((eq & earlier).astype(jnp.int32), axis=1, keepdims=True)
        keep = jnp.transpose((dup == 0).astype(jnp.float32))
        possum = jnp.sum(xp[...] * keep)
        dall = dd[...]
        cmask = jax.lax.broadcasted_iota(jnp.int32, (H, W, 84), 2) >= 4
        ssum = jnp.sum(jnp.where(cmask, dall, 0.0))
        loss_cls = loss_cls + (ssum - possum) * jnp.float32(1.0 / (B * H * W * NCLS))
        tbxs = tbx * W
        tbys = tby * H
        tbws = tbw * W
        tbhs = tbh * H
        cx1 = tbxs - tbws / 2
        cy1 = tbys - tbhs / 2
        cx2 = tbxs + tbws / 2
        cy2 = tbys + tbhs / 2
        for j, corner in enumerate((cx1, cy1, cx2, cy2)):
            ccl = jnp.clip(corner, 0.0, float(RMAX - 1))
            tgt = jnp.clip(jnp.round(ccl), 0.0, float(RMAX - 1)).astype(jnp.int32)
            logits = gd[:, 16 * j:16 * j + 16, :]
            m = jnp.max(logits, axis=1, keepdims=True)
            se = jnp.sum(jnp.exp(logits - m), axis=1, keepdims=True)
            lse = jnp.log(se) + m
            krow = jax.lax.broadcasted_iota(jnp.int32, (NG, RMAX, 16), 1)
            lt = jnp.sum(jnp.where(krow == tgt[:, None, :], logits, 0.0),
                         axis=1, keepdims=True)
            loss_dfl = loss_dfl + jnp.sum(lse - lt)
    loss_dfl = loss_dfl * jnp.float32(1.0 / (N * 4))
    lb = loss_box * BOX_GAIN
    lc = loss_cls * CLS_GAIN
    ld = loss_dfl * DFL_GAIN
    tot = lb + lc + ld
    lane = jax.lax.broadcasted_iota(jnp.int32, (1, 4), 1)
    o[...] = jnp.where(lane == 0, tot,
                       jnp.where(lane == 1, lb, jnp.where(lane == 2, lc, ld)))


NG = N // 16
GJMAX = (8, 2, 1)


def _sc_gather_body(tt, t0, t1, t2,
                    gp0, gp1, gp2, xp0, xp1, xp2, gd0, gd1, gd2,
                    tv, ix0, ix1, ix2, rd0, rd1, rd2,
                    outp, outd, sem):
    wid = lax.axis_index("c") * 16 + lax.axis_index("s")

    @pl.when(wid < NG)
    def _():
        g16 = wid * 16
        for j in range(6):
            pltpu.sync_copy(tt.at[j, pl.ds(g16, 16)], tv.at[j])
        bi = tv[0].astype(jnp.int32)
        ci = tv[1].astype(jnp.int32)
        xv = tv[2]
        yv = tv[3]
        iota16 = lax.iota(jnp.int32, 16)
        tabs = (t0, t1, t2)
        ixrefs = (ix0, ix1, ix2)
        rdrefs = (rd0, rd1, rd2)
        gprefs = (gp0, gp1, gp2)
        xprefs = (xp0, xp1, xp2)
        gdrefs = (gd0, gd1, gd2)
        copies = []
        for s in range(3):
            H, W = SHAPES[s]
            stride = STRIDES[s]
            g0 = xv * jnp.float32(W / stride)
            g1 = yv * jnp.float32(H / stride)
            gi = g0.astype(jnp.int32)
            gj = g1.astype(jnp.int32)
            gm = GJMAX[s]
            ixrefs[s][...] = (bi * gm + gj) * gm + gi
            copies.append(pltpu.async_copy(
                tabs[s].at[ixrefs[s]], rdrefs[s], sem))
        for cp in copies:
            cp.wait()
        for s in range(3):
            rd = rdrefs[s]
            for c in range(4):
                outp[c] = plsc.load_gather(
                    rd, [iota16, jnp.full((16,), 64 + c, jnp.int32)])
            outp[4] = plsc.load_gather(rd, [iota16, 68 + ci])
            for c in range(5):
                pltpu.sync_copy(outp.at[c],
                                gprefs[s].at[pl.ds(c * N + g16, 16)])
            pltpu.sync_copy(outp.at[4], xprefs[s].at[pl.ds(g16, 16)])
            for c in range(64):
                outd[pl.ds(c * 16, 16)] = plsc.load_gather(
                    rd, [iota16, jnp.full((16,), c, jnp.int32)])
            pltpu.sync_copy(outd, gdrefs[s].at[pl.ds(wid * 1024, 1024)])


def _sc_gather(tt, t0, t1, t2):
    f32 = jnp.float32
    cp = pltpu.CompilerParams()
    fields = pltpu.CompilerParams.__dataclass_fields__
    if "needs_layout_passes" in fields:
        cp = dataclasses.replace(cp, needs_layout_passes=False)
    return pl.kernel(
        _sc_gather_body,
        compiler_params=cp,
        out_type=(
            jax.ShapeDtypeStruct((5 * N,), f32),
            jax.ShapeDtypeStruct((5 * N,), f32),
            jax.ShapeDtypeStruct((5 * N,), f32),
            jax.ShapeDtypeStruct((N,), f32),
            jax.ShapeDtypeStruct((N,), f32),
            jax.ShapeDtypeStruct((N,), f32),
            jax.ShapeDtypeStruct((64 * N,), f32),
            jax.ShapeDtypeStruct((64 * N,), f32),
            jax.ShapeDtypeStruct((64 * N,), f32),
        ),
        mesh=plsc.VectorSubcoreMesh(core_axis_name="c", subcore_axis_name="s"),
        scratch_types=[
            pltpu.VMEM((8, 16), f32),
            pltpu.VMEM((16,), jnp.int32),
            pltpu.VMEM((16,), jnp.int32),
            pltpu.VMEM((16,), jnp.int32),
            pltpu.VMEM((16, 256), f32),
            pltpu.VMEM((16, 256), f32),
            pltpu.VMEM((16, 256), f32),
            pltpu.VMEM((8, 16), f32),
            pltpu.VMEM((1024,), f32),
            pltpu.SemaphoreType.DMA,
        ],
    )(tt, t0, t1, t2)


def kernel(pred0, pred1, pred2, dfl0, dfl1, dfl2, targets):
    p0t = jnp.transpose(pred0, (0, 2, 3, 1))
    p1t = jnp.transpose(pred1, (0, 2, 3, 1))
    p2t = jnp.transpose(pred2, (0, 2, 3, 1))
    d1t = jnp.transpose(dfl1, (0, 2, 3, 1))
    d2t = jnp.transpose(dfl2, (0, 2, 3, 1))
    tt = targets.T

    d0a, d1a, d2a, t0, t1, t2 = pl.pallas_call(
        _dense_body,
        grid=(B,),
        in_specs=[
            pl.BlockSpec((1, 64, 64, 84), lambda b: (b, 0, 0, 0)),
            pl.BlockSpec((1, 32, 32, 84), lambda b: (b, 0, 0, 0)),
            pl.BlockSpec((1, 16, 16, 84), lambda b: (b, 0, 0, 0)),
            pl.BlockSpec((1, 64, 8, 64), lambda b: (b, 0, 0, 0)),
            pl.BlockSpec((1, 8, 32, 64), lambda b: (b, 0, 0, 0)),
            pl.BlockSpec((1, 8, 16, 64), lambda b: (b, 0, 0, 0)),
        ],
        out_specs=[
            pl.BlockSpec((64, 64, 84), lambda b: (0, 0, 0)),
            pl.BlockSpec((32, 32, 84), lambda b: (0, 0, 0)),
            pl.BlockSpec((16, 16, 84), lambda b: (0, 0, 0)),
            pl.BlockSpec((1, 64, 256), lambda b: (b, 0, 0)),
            pl.BlockSpec((1, 4, 256), lambda b: (b, 0, 0)),
            pl.BlockSpec((1, 1, 256), lambda b: (b, 0, 0)),
        ],
        out_shape=[
            jax.ShapeDtypeStruct((64, 64, 84), jnp.float32),
            jax.ShapeDtypeStruct((32, 32, 84), jnp.float32),
            jax.ShapeDtypeStruct((16, 16, 84), jnp.float32),
            jax.ShapeDtypeStruct((B, 64, 256), jnp.float32),
            jax.ShapeDtypeStruct((B, 4, 256), jnp.float32),
            jax.ShapeDtypeStruct((B, 1, 256), jnp.float32),
        ],
        interpret=_INTERPRET,
    )(p0t, p1t, p2t, dfl0, d1t, d2t)

    gp0, gp1, gp2, xp0, xp1, xp2, gd0, gd1, gd2 = _sc_gather(
        tt, t0.reshape(B * 64, 256), t1.reshape(B * 4, 256),
        t2.reshape(B, 256))
    gp0, gp1, gp2 = (g.reshape(5, NG, 16) for g in (gp0, gp1, gp2))
    xp0, xp1, xp2 = (xg.reshape(1, N) for xg in (xp0, xp1, xp2))
    gd0, gd1, gd2 = (g.reshape(NG, 64, 16) for g in (gd0, gd1, gd2))
    tt3 = tt.reshape(6, NG, 16)

    out = pl.pallas_call(
        _combine_body,
        out_shape=jax.ShapeDtypeStruct((1, 4), jnp.float32),
        interpret=_INTERPRET,
    )(tt, tt3, gp0, gp1, gp2, xp0, xp1, xp2, gd0, gd1, gd2, d0a, d1a, d2a)
    return out.reshape(4)


# speedup vs baseline: 5.0535x; 1.2047x over previous
import dataclasses

import jax
import jax.numpy as jnp
from jax import lax
from jax.experimental import pallas as pl
from jax.experimental.pallas import tpu as pltpu
from jax.experimental.pallas import tpu_sc as plsc

NCLS = 80
RMAX = 16
BOX_GAIN, CLS_GAIN, DFL_GAIN = 7.5, 0.5, 1.5
STRIDES = (8.0, 16.0, 32.0)
EPS = 1e-07
B = 32
N = 400
SHAPES = ((64, 64), (32, 32), (16, 16))

_INTERPRET = False


def _dense_body(p0, p1, p2, o0, o1, o2):
    i = pl.program_id(0)
    for s, (ref, o) in enumerate(((p0, o0), (p1, o1), (p2, o2))):
        H, W = SHAPES[s]

        @pl.when(i == 0)
        def _():
            o[...] = jnp.zeros_like(o)

        for h in range(0, H, 4):
            x = ref[0, h:h + 4]
            ax = jnp.abs(x)
            f = 0.5 * (x + ax) + jnp.log(1.0 + jnp.exp(-ax))
            o[h:h + 4] += f


def _prep_body(p0, d0, p1, d1, p2, d2, t0, t1, t2):
    dt0 = jnp.transpose(d0[:, :, :, 0:8], (0, 2, 3, 1)).reshape(B * 64, 64)
    t0[...] = jnp.concatenate(
        [dt0, p0[...].reshape(B * 64, 84),
         jnp.zeros((B * 64, 108), jnp.float32)], axis=1)
    t1[...] = jnp.concatenate(
        [d1[:, :, 0:2, :].reshape(B * 4, 64),
         p1[:, :, 0:2, :].reshape(B * 4, 84),
         jnp.zeros((B * 4, 108), jnp.float32)], axis=1)
    t2[...] = jnp.concatenate(
        [d2[:, :, 0:1, :].reshape(B, 64), p2[:, :, 0:1, :].reshape(B, 84),
         jnp.zeros((B, 108), jnp.float32)], axis=1)


def _iou(px, py, pw, ph, tx, ty, tw, th):
    b1x1 = px - pw / 2
    b1x2 = px + pw / 2
    b1y1 = py - ph / 2
    b1y2 = py + ph / 2
    b2x1 = tx - tw / 2
    b2x2 = tx + tw / 2
    b2y1 = ty - th / 2
    b2y2 = ty + th / 2
    inter = (jnp.clip(jnp.minimum(b1x2, b2x2) - jnp.maximum(b1x1, b2x1), 0, None)
             * jnp.clip(jnp.minimum(b1y2, b2y2) - jnp.maximum(b1y1, b2y1), 0, None))
    w1, h1 = b1x2 - b1x1, b1y2 - b1y1 + EPS
    w2, h2 = b2x2 - b2x1, b2y2 - b2y1 + EPS
    union = w1 * h1 + w2 * h2 - inter + EPS
    return inter / union


def _combine_body(tt2, tt3, gp0, gp1, gp2, xp0, xp1, xp2, gd0, gd1, gd2,
                  d0, d1, d2, o):
    bi2 = tt2[0:1, :].astype(jnp.int32)
    ci2 = tt2[1:2, :].astype(jnp.int32)
    x2t = tt2[2:3, :]
    y2t = tt2[3:4, :]
    x3 = tt3[2]
    y3 = tt3[3]
    w3 = tt3[4]
    h3 = tt3[5]
    loss_box = jnp.float32(0.0)
    loss_cls = jnp.float32(0.0)
    loss_dfl = jnp.float32(0.0)
    for s, (gp, xp, gd, dd) in enumerate(((gp0, xp0, gd0, d0),
                                          (gp1, xp1, gd1, d1),
                                          (gp2, xp2, gd2, d2))):
        H, W = SHAPES[s]
        stride = STRIDES[s]
        sw = jnp.float32(W / stride)
        sh = jnp.float32(H / stride)
        g0 = x3 * sw
        g1 = y3 * sh
        gif = jnp.floor(g0)
        gjf = jnp.floor(g1)
        tbx = g0 - gif
        tby = g1 - gjf
        tbw = w3 * sw
        tbh = h3 * sh
        iou = _iou(gp[:, 0, :], gp[:, 1, :], gp[:, 2, :], gp[:, 3, :],
                   tbx, tby, tbw, tbh)
        loss_box = loss_box + jnp.sum(1.0 - iou) * jnp.float32(1.0 / N)
        gi2 = jnp.floor(x2t * sw).astype(jnp.int32)
        gj2 = jnp.floor(y2t * sh).astype(jnp.int32)
        flat = bi2 * (H * W) + gj2 * W + gi2
        key = flat * NCLS + ci2
        keyc = jnp.transpose(key)
        eq = (keyc == key)
        earlier = (jax.lax.broadcasted_iota(jnp.int32, (N, N), 1)
                   < jax.lax.broadcasted_iota(jnp.int32, (N, N), 0))
        dup = jnp.sum((eq & earlier).astype(jnp.int32), axis=1, keepdims=True)
        keep = jnp.transpose((dup == 0).astype(jnp.float32))
        possum = jnp.sum(xp[...].reshape(1, N) * keep)
        dall = dd[...]
        cmask = jax.lax.broadcasted_iota(jnp.int32, (H, W, 84), 2) >= 4
        ssum = jnp.sum(jnp.where(cmask, dall, 0.0))
        loss_cls = loss_cls + (ssum - possum) * jnp.float32(1.0 / (B * H * W * NCLS))
        tbxs = tbx * W
        tbys = tby * H
        tbws = tbw * W
        tbhs = tbh * H
        cx1 = tbxs - tbws / 2
        cy1 = tbys - tbhs / 2
        cx2 = tbxs + tbws / 2
        cy2 = tbys + tbhs / 2
        for j, corner in enumerate((cx1, cy1, cx2, cy2)):
            ccl = jnp.clip(corner, 0.0, float(RMAX - 1))
            tgt = jnp.clip(jnp.round(ccl), 0.0, float(RMAX - 1)).astype(jnp.int32)
            logits = gd[:, 16 * j:16 * j + 16, :]
            m = jnp.max(logits, axis=1, keepdims=True)
            se = jnp.sum(jnp.exp(logits - m), axis=1, keepdims=True)
            lse = jnp.log(se) + m
            krow = jax.lax.broadcasted_iota(jnp.int32, (NG, RMAX, 16), 1)
            lt = jnp.sum(jnp.where(krow == tgt[:, None, :], logits, 0.0),
                         axis=1, keepdims=True)
            loss_dfl = loss_dfl + jnp.sum(lse - lt)
    loss_dfl = loss_dfl * jnp.float32(1.0 / (N * 4))
    lb = loss_box * BOX_GAIN
    lc = loss_cls * CLS_GAIN
    ld = loss_dfl * DFL_GAIN
    tot = lb + lc + ld
    lane = jax.lax.broadcasted_iota(jnp.int32, (1, 4), 1)
    o[...] = jnp.where(lane == 0, tot,
                       jnp.where(lane == 1, lb, jnp.where(lane == 2, lc, ld)))


NG = N // 16
GJMAX = (8, 2, 1)


def _sc_gather_body(tt, t0, t1, t2,
                    gp0, gp1, gp2, xp0, xp1, xp2, gd0, gd1, gd2,
                    tv, ix0, ix1, ix2, rd0, rd1, rd2,
                    outp, outd, sem):
    wid = lax.axis_index("c") * 16 + lax.axis_index("s")

    @pl.when(wid < NG)
    def _():
        g16 = wid * 16
        for j in range(6):
            pltpu.sync_copy(tt.at[j, pl.ds(g16, 16)], tv.at[j])
        bi = tv[0].astype(jnp.int32)
        ci = tv[1].astype(jnp.int32)
        xv = tv[2]
        yv = tv[3]
        iota16 = lax.iota(jnp.int32, 16)
        tabs = (t0, t1, t2)
        ixrefs = (ix0, ix1, ix2)
        rdrefs = (rd0, rd1, rd2)
        gprefs = (gp0, gp1, gp2)
        xprefs = (xp0, xp1, xp2)
        gdrefs = (gd0, gd1, gd2)
        copies = []
        for s in range(3):
            H, W = SHAPES[s]
            stride = STRIDES[s]
            g0 = xv * jnp.float32(W / stride)
            g1 = yv * jnp.float32(H / stride)
            gi = g0.astype(jnp.int32)
            gj = g1.astype(jnp.int32)
            gm = GJMAX[s]
            ixrefs[s][...] = (bi * gm + gj) * gm + gi
            copies.append(pltpu.async_copy(
                tabs[s].at[ixrefs[s]], rdrefs[s], sem))
        for cp in copies:
            cp.wait()
        for s in range(3):
            rd = rdrefs[s]
            for c in range(4):
                outp[c] = plsc.load_gather(
                    rd, [iota16, jnp.full((16,), 64 + c, jnp.int32)])
            outp[4] = plsc.load_gather(rd, [iota16, 68 + ci])
            pltpu.sync_copy(outp.at[0:5], gprefs[s].at[wid])
            pltpu.sync_copy(outp.at[4], xprefs[s].at[pl.ds(g16, 16)])
            for c in range(64):
                outd[c] = plsc.load_gather(
                    rd, [iota16, jnp.full((16,), c, jnp.int32)])
            pltpu.sync_copy(outd, gdrefs[s].at[wid])


def _sc_gather(tt, t0, t1, t2):
    f32 = jnp.float32
    cp = pltpu.CompilerParams()
    fields = pltpu.CompilerParams.__dataclass_fields__
    if "needs_layout_passes" in fields:
        cp = dataclasses.replace(cp, needs_layout_passes=False)
    return pl.kernel(
        _sc_gather_body,
        compiler_params=cp,
        out_type=(
            jax.ShapeDtypeStruct((NG, 5, 16), f32),
            jax.ShapeDtypeStruct((NG, 5, 16), f32),
            jax.ShapeDtypeStruct((NG, 5, 16), f32),
            jax.ShapeDtypeStruct((N,), f32),
            jax.ShapeDtypeStruct((N,), f32),
            jax.ShapeDtypeStruct((N,), f32),
            jax.ShapeDtypeStruct((NG, 64, 16), f32),
            jax.ShapeDtypeStruct((NG, 64, 16), f32),
            jax.ShapeDtypeStruct((NG, 64, 16), f32),
        ),
        mesh=plsc.VectorSubcoreMesh(core_axis_name="c", subcore_axis_name="s"),
        scratch_types=[
            pltpu.VMEM((8, 16), f32),
            pltpu.VMEM((16,), jnp.int32),
            pltpu.VMEM((16,), jnp.int32),
            pltpu.VMEM((16,), jnp.int32),
            pltpu.VMEM((16, 256), f32),
            pltpu.VMEM((16, 256), f32),
            pltpu.VMEM((16, 256), f32),
            pltpu.VMEM((8, 16), f32),
            pltpu.VMEM((64, 16), f32),
            pltpu.SemaphoreType.DMA,
        ],
    )(tt, t0, t1, t2)


def kernel(pred0, pred1, pred2, dfl0, dfl1, dfl2, targets):
    p0t = jnp.transpose(pred0, (0, 2, 3, 1))
    p1t = jnp.transpose(pred1, (0, 2, 3, 1))
    p2t = jnp.transpose(pred2, (0, 2, 3, 1))
    d1t = jnp.transpose(dfl1, (0, 2, 3, 1))
    d2t = jnp.transpose(dfl2, (0, 2, 3, 1))
    tt = targets.T

    t0, t1, t2 = pl.pallas_call(
        _prep_body,
        grid=(1,),
        in_specs=[
            pl.BlockSpec((B, 8, 8, 84), lambda i: (0, 0, 0, 0)),
            pl.BlockSpec((B, 64, 8, 64), lambda i: (0, 0, 0, 0)),
            pl.BlockSpec((B, 2, 32, 84), lambda i: (0, 0, 0, 0)),
            pl.BlockSpec((B, 2, 32, 64), lambda i: (0, 0, 0, 0)),
            pl.BlockSpec((B, 1, 16, 84), lambda i: (0, 0, 0, 0)),
            pl.BlockSpec((B, 1, 16, 64), lambda i: (0, 0, 0, 0)),
        ],
        out_specs=[
            pl.BlockSpec((B * 64, 256), lambda i: (0, 0)),
            pl.BlockSpec((B * 4, 256), lambda i: (0, 0)),
            pl.BlockSpec((B, 256), lambda i: (0, 0)),
        ],
        out_shape=[
            jax.ShapeDtypeStruct((B * 64, 256), jnp.float32),
            jax.ShapeDtypeStruct((B * 4, 256), jnp.float32),
            jax.ShapeDtypeStruct((B, 256), jnp.float32),
        ],
        interpret=_INTERPRET,
    )(p0t, dfl0, p1t, d1t, p2t, d2t)

    gp0, gp1, gp2, xp0, xp1, xp2, gd0, gd1, gd2 = _sc_gather(tt, t0, t1, t2)
    tt3 = tt.reshape(6, NG, 16)

    d0a, d1a, d2a = pl.pallas_call(
        _dense_body,
        grid=(B,),
        in_specs=[
            pl.BlockSpec((1, 64, 64, 84), lambda b: (b, 0, 0, 0)),
            pl.BlockSpec((1, 32, 32, 84), lambda b: (b, 0, 0, 0)),
            pl.BlockSpec((1, 16, 16, 84), lambda b: (b, 0, 0, 0)),
        ],
        out_specs=[
            pl.BlockSpec((64, 64, 84), lambda b: (0, 0, 0)),
            pl.BlockSpec((32, 32, 84), lambda b: (0, 0, 0)),
            pl.BlockSpec((16, 16, 84), lambda b: (0, 0, 0)),
        ],
        out_shape=[
            jax.ShapeDtypeStruct((64, 64, 84), jnp.float32),
            jax.ShapeDtypeStruct((32, 32, 84), jnp.float32),
            jax.ShapeDtypeStruct((16, 16, 84), jnp.float32),
        ],
        interpret=_INTERPRET,
    )(p0t, p1t, p2t)

    out = pl.pallas_call(
        _combine_body,
        out_shape=jax.ShapeDtypeStruct((1, 4), jnp.float32),
        interpret=_INTERPRET,
    )(tt, tt3, gp0, gp1, gp2, xp0, xp1, xp2, gd0, gd1, gd2, d0a, d1a, d2a)
    return out.reshape(4)


# speedup vs baseline: 5.1630x; 1.0217x over previous
import dataclasses

import jax
import jax.numpy as jnp
from jax import lax
from jax.experimental import pallas as pl
from jax.experimental.pallas import tpu as pltpu
from jax.experimental.pallas import tpu_sc as plsc

NCLS = 80
RMAX = 16
BOX_GAIN, CLS_GAIN, DFL_GAIN = 7.5, 0.5, 1.5
STRIDES = (8.0, 16.0, 32.0)
EPS = 1e-07
B = 32
N = 400
SHAPES = ((64, 64), (32, 32), (16, 16))

_INTERPRET = False


def _dense_body(p0, p1, p2, o8, a0, a1, a2):
    i = pl.program_id(0)
    accs = (a0, a1, a2)
    for s, ref in enumerate((p0, p1, p2)):
        H, W = SHAPES[s]
        o = accs[s]

        @pl.when(i == 0)
        def _():
            o[...] = jnp.zeros_like(o)

        for h in range(0, H, 4):
            x = ref[0, h:h + 4]
            ax = jnp.abs(x)
            f = 0.5 * (x + ax) + jnp.log(1.0 + jnp.exp(-ax))
            o[h:h + 4] += f

    @pl.when(i == B - 1)
    def _():
        sums = []
        for s in range(3):
            H, W = SHAPES[s]
            cmask = jax.lax.broadcasted_iota(jnp.int32, (H, W, 84), 2) >= 4
            sums.append(jnp.sum(jnp.where(cmask, accs[s][...], 0.0)))
        lane = jax.lax.broadcasted_iota(jnp.int32, (1, 8), 1)
        o8[...] = jnp.where(lane == 0, sums[0],
                            jnp.where(lane == 1, sums[1],
                                      jnp.where(lane == 2, sums[2], 0.0)))


def _prep_body(p0, d0, p1, d1, p2, d2, t0, t1, t2):
    dt0 = jnp.transpose(d0[:, :, :, 0:8], (0, 2, 3, 1)).reshape(B * 64, 64)
    t0[...] = jnp.concatenate(
        [dt0, p0[...].reshape(B * 64, 84),
         jnp.zeros((B * 64, 108), jnp.float32)], axis=1)
    t1[...] = jnp.concatenate(
        [d1[:, :, 0:2, :].reshape(B * 4, 64),
         p1[:, :, 0:2, :].reshape(B * 4, 84),
         jnp.zeros((B * 4, 108), jnp.float32)], axis=1)
    t2[...] = jnp.concatenate(
        [d2[:, :, 0:1, :].reshape(B, 64), p2[:, :, 0:1, :].reshape(B, 84),
         jnp.zeros((B, 108), jnp.float32)], axis=1)


def _iou(px, py, pw, ph, tx, ty, tw, th):
    b1x1 = px - pw / 2
    b1x2 = px + pw / 2
    b1y1 = py - ph / 2
    b1y2 = py + ph / 2
    b2x1 = tx - tw / 2
    b2x2 = tx + tw / 2
    b2y1 = ty - th / 2
    b2y2 = ty + th / 2
    inter = (jnp.clip(jnp.minimum(b1x2, b2x2) - jnp.maximum(b1x1, b2x1), 0, None)
             * jnp.clip(jnp.minimum(b1y2, b2y2) - jnp.maximum(b1y1, b2y1), 0, None))
    w1, h1 = b1x2 - b1x1, b1y2 - b1y1 + EPS
    w2, h2 = b2x2 - b2x1, b2y2 - b2y1 + EPS
    union = w1 * h1 + w2 * h2 - inter + EPS
    return inter / union


def _combine_body(tt2, tt3, gp0, gp1, gp2, xp0, xp1, xp2, gd0, gd1, gd2,
                  ds, o):
    bi2 = tt2[0:1, :].astype(jnp.int32)
    ci2 = tt2[1:2, :].astype(jnp.int32)
    x2t = tt2[2:3, :]
    y2t = tt2[3:4, :]
    x3 = tt3[2]
    y3 = tt3[3]
    w3 = tt3[4]
    h3 = tt3[5]
    loss_box = jnp.float32(0.0)
    loss_cls = jnp.float32(0.0)
    loss_dfl = jnp.float32(0.0)
    for s, (gp, xp, gd) in enumerate(((gp0, xp0, gd0), (gp1, xp1, gd1),
                                      (gp2, xp2, gd2))):
        H, W = SHAPES[s]
        stride = STRIDES[s]
        sw = jnp.float32(W / stride)
        sh = jnp.float32(H / stride)
        g0 = x3 * sw
        g1 = y3 * sh
        gif = jnp.floor(g0)
        gjf = jnp.floor(g1)
        tbx = g0 - gif
        tby = g1 - gjf
        tbw = w3 * sw
        tbh = h3 * sh
        iou = _iou(gp[:, 0, :], gp[:, 1, :], gp[:, 2, :], gp[:, 3, :],
                   tbx, tby, tbw, tbh)
        loss_box = loss_box + jnp.sum(1.0 - iou) * jnp.float32(1.0 / N)
        gi2 = jnp.floor(x2t * sw).astype(jnp.int32)
        gj2 = jnp.floor(y2t * sh).astype(jnp.int32)
        flat = bi2 * (H * W) + gj2 * W + gi2
        key = flat * NCLS + ci2
        keyc = jnp.transpose(key)
        eq = (keyc == key)
        earlier = (jax.lax.broadcasted_iota(jnp.int32, (N, N), 1)
                   < jax.lax.broadcasted_iota(jnp.int32, (N, N), 0))
        dup = jnp.sum((eq & earlier).astype(jnp.int32), axis=1, keepdims=True)
        keep = jnp.transpose((dup == 0).astype(jnp.float32))
        possum = jnp.sum(xp[...].reshape(1, N) * keep)
        loss_cls = loss_cls + (ds[0, s] - possum) * jnp.float32(1.0 / (B * H * W * NCLS))
        tbxs = tbx * W
        tbys = tby * H
        tbws = tbw * W
        tbhs = tbh * H
        cx1 = tbxs - tbws / 2
        cy1 = tbys - tbhs / 2
        cx2 = tbxs + tbws / 2
        cy2 = tbys + tbhs / 2
        for j, corner in enumerate((cx1, cy1, cx2, cy2)):
            ccl = jnp.clip(corner, 0.0, float(RMAX - 1))
            tgt = jnp.clip(jnp.round(ccl), 0.0, float(RMAX - 1)).astype(jnp.int32)
            logits = gd[:, 16 * j:16 * j + 16, :]
            m = jnp.max(logits, axis=1, keepdims=True)
            se = jnp.sum(jnp.exp(logits - m), axis=1, keepdims=True)
            lse = jnp.log(se) + m
            krow = jax.lax.broadcasted_iota(jnp.int32, (NG, RMAX, 16), 1)
            lt = jnp.sum(jnp.where(krow == tgt[:, None, :], logits, 0.0),
                         axis=1, keepdims=True)
            loss_dfl = loss_dfl + jnp.sum(lse - lt)
    loss_dfl = loss_dfl * jnp.float32(1.0 / (N * 4))
    lb = loss_box * BOX_GAIN
    lc = loss_cls * CLS_GAIN
    ld = loss_dfl * DFL_GAIN
    tot = lb + lc + ld
    lane = jax.lax.broadcasted_iota(jnp.int32, (1, 4), 1)
    o[...] = jnp.where(lane == 0, tot,
                       jnp.where(lane == 1, lb, jnp.where(lane == 2, lc, ld)))


NG = N // 16
GJMAX = (8, 2, 1)


def _sc_gather_body(tt, t0, t1, t2,
                    gp0, gp1, gp2, xp0, xp1, xp2, gd0, gd1, gd2,
                    tv, ix0, ix1, ix2, rd0, rd1, rd2,
                    outp, outd, sem):
    wid = lax.axis_index("c") * 16 + lax.axis_index("s")

    @pl.when(wid < NG)
    def _():
        g16 = wid * 16
        for j in range(6):
            pltpu.sync_copy(tt.at[j, pl.ds(g16, 16)], tv.at[j])
        bi = tv[0].astype(jnp.int32)
        ci = tv[1].astype(jnp.int32)
        xv = tv[2]
        yv = tv[3]
        iota16 = lax.iota(jnp.int32, 16)
        tabs = (t0, t1, t2)
        ixrefs = (ix0, ix1, ix2)
        rdrefs = (rd0, rd1, rd2)
        gprefs = (gp0, gp1, gp2)
        xprefs = (xp0, xp1, xp2)
        gdrefs = (gd0, gd1, gd2)
        copies = []
        for s in range(3):
            H, W = SHAPES[s]
            stride = STRIDES[s]
            g0 = xv * jnp.float32(W / stride)
            g1 = yv * jnp.float32(H / stride)
            gi = g0.astype(jnp.int32)
            gj = g1.astype(jnp.int32)
            gm = GJMAX[s]
            ixrefs[s][...] = (bi * gm + gj) * gm + gi
            copies.append(pltpu.async_copy(
                tabs[s].at[ixrefs[s]], rdrefs[s], sem))
        for cp in copies:
            cp.wait()
        for s in range(3):
            rd = rdrefs[s]
            for c in range(4):
                outp[c] = plsc.load_gather(
                    rd, [iota16, jnp.full((16,), 64 + c, jnp.int32)])
            outp[4] = plsc.load_gather(rd, [iota16, 68 + ci])
            pltpu.sync_copy(outp.at[0:5], gprefs[s].at[wid])
            pltpu.sync_copy(outp.at[4], xprefs[s].at[pl.ds(g16, 16)])
            for c in range(64):
                outd[c] = plsc.load_gather(
                    rd, [iota16, jnp.full((16,), c, jnp.int32)])
            pltpu.sync_copy(outd, gdrefs[s].at[wid])


def _sc_gather(tt, t0, t1, t2):
    f32 = jnp.float32
    cp = pltpu.CompilerParams()
    fields = pltpu.CompilerParams.__dataclass_fields__
    if "needs_layout_passes" in fields:
        cp = dataclasses.replace(cp, needs_layout_passes=False)
    return pl.kernel(
        _sc_gather_body,
        compiler_params=cp,
        out_type=(
            jax.ShapeDtypeStruct((NG, 5, 16), f32),
            jax.ShapeDtypeStruct((NG, 5, 16), f32),
            jax.ShapeDtypeStruct((NG, 5, 16), f32),
            jax.ShapeDtypeStruct((N,), f32),
            jax.ShapeDtypeStruct((N,), f32),
            jax.ShapeDtypeStruct((N,), f32),
            jax.ShapeDtypeStruct((NG, 64, 16), f32),
            jax.ShapeDtypeStruct((NG, 64, 16), f32),
            jax.ShapeDtypeStruct((NG, 64, 16), f32),
        ),
        mesh=plsc.VectorSubcoreMesh(core_axis_name="c", subcore_axis_name="s"),
        scratch_types=[
            pltpu.VMEM((8, 16), f32),
            pltpu.VMEM((16,), jnp.int32),
            pltpu.VMEM((16,), jnp.int32),
            pltpu.VMEM((16,), jnp.int32),
            pltpu.VMEM((16, 256), f32),
            pltpu.VMEM((16, 256), f32),
            pltpu.VMEM((16, 256), f32),
            pltpu.VMEM((8, 16), f32),
            pltpu.VMEM((64, 16), f32),
            pltpu.SemaphoreType.DMA,
        ],
    )(tt, t0, t1, t2)


def kernel(pred0, pred1, pred2, dfl0, dfl1, dfl2, targets):
    p0t = jnp.transpose(pred0, (0, 2, 3, 1))
    p1t = jnp.transpose(pred1, (0, 2, 3, 1))
    p2t = jnp.transpose(pred2, (0, 2, 3, 1))
    d1t = jnp.transpose(dfl1, (0, 2, 3, 1))
    d2t = jnp.transpose(dfl2, (0, 2, 3, 1))
    tt = targets.T

    t0, t1, t2 = pl.pallas_call(
        _prep_body,
        grid=(1,),
        in_specs=[
            pl.BlockSpec((B, 8, 8, 84), lambda i: (0, 0, 0, 0)),
            pl.BlockSpec((B, 64, 8, 64), lambda i: (0, 0, 0, 0)),
            pl.BlockSpec((B, 2, 32, 84), lambda i: (0, 0, 0, 0)),
            pl.BlockSpec((B, 2, 32, 64), lambda i: (0, 0, 0, 0)),
            pl.BlockSpec((B, 1, 16, 84), lambda i: (0, 0, 0, 0)),
            pl.BlockSpec((B, 1, 16, 64), lambda i: (0, 0, 0, 0)),
        ],
        out_specs=[
            pl.BlockSpec((B * 64, 256), lambda i: (0, 0)),
            pl.BlockSpec((B * 4, 256), lambda i: (0, 0)),
            pl.BlockSpec((B, 256), lambda i: (0, 0)),
        ],
        out_shape=[
            jax.ShapeDtypeStruct((B * 64, 256), jnp.float32),
            jax.ShapeDtypeStruct((B * 4, 256), jnp.float32),
            jax.ShapeDtypeStruct((B, 256), jnp.float32),
        ],
        interpret=_INTERPRET,
    )(p0t, dfl0, p1t, d1t, p2t, d2t)

    gp0, gp1, gp2, xp0, xp1, xp2, gd0, gd1, gd2 = _sc_gather(tt, t0, t1, t2)
    tt3 = tt.reshape(6, NG, 16)

    ds = pl.pallas_call(
        _dense_body,
        grid=(B,),
        in_specs=[
            pl.BlockSpec((1, 64, 64, 84), lambda b: (b, 0, 0, 0)),
            pl.BlockSpec((1, 32, 32, 84), lambda b: (b, 0, 0, 0)),
            pl.BlockSpec((1, 16, 16, 84), lambda b: (b, 0, 0, 0)),
        ],
        out_specs=pl.BlockSpec((1, 8), lambda b: (0, 0)),
        out_shape=jax.ShapeDtypeStruct((1, 8), jnp.float32),
        scratch_shapes=[
            pltpu.VMEM((64, 64, 84), jnp.float32),
            pltpu.VMEM((32, 32, 84), jnp.float32),
            pltpu.VMEM((16, 16, 84), jnp.float32),
        ],
        interpret=_INTERPRET,
    )(p0t, p1t, p2t)

    out = pl.pallas_call(
        _combine_body,
        out_shape=jax.ShapeDtypeStruct((1, 4), jnp.float32),
        interpret=_INTERPRET,
    )(tt, tt3, gp0, gp1, gp2, xp0, xp1, xp2, gd0, gd1, gd2, ds)
    return out.reshape(4)


# speedup vs baseline: 5.1648x; 1.0004x over previous
import dataclasses

import jax
import jax.numpy as jnp
from jax import lax
from jax.experimental import pallas as pl
from jax.experimental.pallas import tpu as pltpu
from jax.experimental.pallas import tpu_sc as plsc

NCLS = 80
RMAX = 16
BOX_GAIN, CLS_GAIN, DFL_GAIN = 7.5, 0.5, 1.5
STRIDES = (8.0, 16.0, 32.0)
EPS = 1e-07
B = 32
N = 400
SHAPES = ((64, 64), (32, 32), (16, 16))

def _dense_body(p0, p1, p2, o8, a0, a1, a2):
    i = pl.program_id(0)
    accs = (a0, a1, a2)
    for s, ref in enumerate((p0, p1, p2)):
        H, W = SHAPES[s]
        o = accs[s]

        @pl.when(i == 0)
        def _():
            o[...] = jnp.zeros_like(o)

        for h in range(0, H, 4):
            x = ref[0, h:h + 4]
            ax = jnp.abs(x)
            f = 0.5 * (x + ax) + jnp.log(1.0 + jnp.exp(-ax))
            o[h:h + 4] += f

    @pl.when(i == B - 1)
    def _():
        sums = []
        for s in range(3):
            H, W = SHAPES[s]
            cmask = jax.lax.broadcasted_iota(jnp.int32, (H, W, 84), 2) >= 4
            sums.append(jnp.sum(jnp.where(cmask, accs[s][...], 0.0)))
        lane = jax.lax.broadcasted_iota(jnp.int32, (1, 8), 1)
        o8[...] = jnp.where(lane == 0, sums[0],
                            jnp.where(lane == 1, sums[1],
                                      jnp.where(lane == 2, sums[2], 0.0)))


def _prep_body(p0, d0, p1, d1, p2, d2, t0, t1, t2):
    dt0 = jnp.transpose(d0[:, :, :, 0:8], (0, 2, 3, 1)).reshape(B * 64, 64)
    t0[...] = jnp.concatenate(
        [dt0, p0[...].reshape(B * 64, 84),
         jnp.zeros((B * 64, 108), jnp.float32)], axis=1)
    t1[...] = jnp.concatenate(
        [d1[:, :, 0:2, :].reshape(B * 4, 64),
         p1[:, :, 0:2, :].reshape(B * 4, 84),
         jnp.zeros((B * 4, 108), jnp.float32)], axis=1)
    t2[...] = jnp.concatenate(
        [d2[:, :, 0:1, :].reshape(B, 64), p2[:, :, 0:1, :].reshape(B, 84),
         jnp.zeros((B, 108), jnp.float32)], axis=1)


def _iou(px, py, pw, ph, tx, ty, tw, th):
    b1x1 = px - pw / 2
    b1x2 = px + pw / 2
    b1y1 = py - ph / 2
    b1y2 = py + ph / 2
    b2x1 = tx - tw / 2
    b2x2 = tx + tw / 2
    b2y1 = ty - th / 2
    b2y2 = ty + th / 2
    inter = (jnp.clip(jnp.minimum(b1x2, b2x2) - jnp.maximum(b1x1, b2x1), 0, None)
             * jnp.clip(jnp.minimum(b1y2, b2y2) - jnp.maximum(b1y1, b2y1), 0, None))
    w1, h1 = b1x2 - b1x1, b1y2 - b1y1 + EPS
    w2, h2 = b2x2 - b2x1, b2y2 - b2y1 + EPS
    union = w1 * h1 + w2 * h2 - inter + EPS
    return inter / union


def _combine_body(tt2, tt3, gp0, gp1, gp2, xp0, xp1, xp2, gd0, gd1, gd2,
                  ds, o):
    bi2 = tt2[0:1, :].astype(jnp.int32)
    ci2 = tt2[1:2, :].astype(jnp.int32)
    x2t = tt2[2:3, :]
    y2t = tt2[3:4, :]
    x3 = tt3[2]
    y3 = tt3[3]
    w3 = tt3[4]
    h3 = tt3[5]
    loss_box = jnp.float32(0.0)
    loss_cls = jnp.float32(0.0)
    loss_dfl = jnp.float32(0.0)
    for s, (gp, xp, gd) in enumerate(((gp0, xp0, gd0), (gp1, xp1, gd1),
                                      (gp2, xp2, gd2))):
        H, W = SHAPES[s]
        stride = STRIDES[s]
        sw = jnp.float32(W / stride)
        sh = jnp.float32(H / stride)
        g0 = x3 * sw
        g1 = y3 * sh
        gif = jnp.floor(g0)
        gjf = jnp.floor(g1)
        tbx = g0 - gif
        tby = g1 - gjf
        tbw = w3 * sw
        tbh = h3 * sh
        iou = _iou(gp[:, 0, :], gp[:, 1, :], gp[:, 2, :], gp[:, 3, :],
                   tbx, tby, tbw, tbh)
        loss_box = loss_box + jnp.sum(1.0 - iou) * jnp.float32(1.0 / N)
        gi2 = jnp.floor(x2t * sw).astype(jnp.int32)
        gj2 = jnp.floor(y2t * sh).astype(jnp.int32)
        flat = bi2 * (H * W) + gj2 * W + gi2
        key = flat * NCLS + ci2
        keyc = jnp.transpose(key)
        eq = (keyc == key)
        earlier = (jax.lax.broadcasted_iota(jnp.int32, (N, N), 1)
                   < jax.lax.broadcasted_iota(jnp.int32, (N, N), 0))
        dup = jnp.sum((eq & earlier).astype(jnp.int32), axis=1, keepdims=True)
        keep = jnp.transpose((dup == 0).astype(jnp.float32))
        possum = jnp.sum(xp[...].reshape(1, N) * keep)
        loss_cls = loss_cls + (ds[0, s] - possum) * jnp.float32(1.0 / (B * H * W * NCLS))
        tbxs = tbx * W
        tbys = tby * H
        tbws = tbw * W
        tbhs = tbh * H
        cx1 = tbxs - tbws / 2
        cy1 = tbys - tbhs / 2
        cx2 = tbxs + tbws / 2
        cy2 = tbys + tbhs / 2
        for j, corner in enumerate((cx1, cy1, cx2, cy2)):
            ccl = jnp.clip(corner, 0.0, float(RMAX - 1))
            tgt = jnp.clip(jnp.round(ccl), 0.0, float(RMAX - 1)).astype(jnp.int32)
            logits = gd[:, 16 * j:16 * j + 16, :]
            m = jnp.max(logits, axis=1, keepdims=True)
            se = jnp.sum(jnp.exp(logits - m), axis=1, keepdims=True)
            lse = jnp.log(se) + m
            krow = jax.lax.broadcasted_iota(jnp.int32, (NG, RMAX, 16), 1)
            lt = jnp.sum(jnp.where(krow == tgt[:, None, :], logits, 0.0),
                         axis=1, keepdims=True)
            loss_dfl = loss_dfl + jnp.sum(lse - lt)
    loss_dfl = loss_dfl * jnp.float32(1.0 / (N * 4))
    lb = loss_box * BOX_GAIN
    lc = loss_cls * CLS_GAIN
    ld = loss_dfl * DFL_GAIN
    tot = lb + lc + ld
    lane = jax.lax.broadcasted_iota(jnp.int32, (1, 4), 1)
    o[...] = jnp.where(lane == 0, tot,
                       jnp.where(lane == 1, lb, jnp.where(lane == 2, lc, ld)))


NG = N // 16
GJMAX = (8, 2, 1)


def _sc_gather_body(tt, t0, t1, t2,
                    gp0, gp1, gp2, xp0, xp1, xp2, gd0, gd1, gd2,
                    tv, ix0, ix1, ix2, rd0, rd1, rd2,
                    outp, outd, sem):
    wid = lax.axis_index("c") * 16 + lax.axis_index("s")

    @pl.when(wid < NG)
    def _():
        g16 = wid * 16
        for j in range(6):
            pltpu.sync_copy(tt.at[j, pl.ds(g16, 16)], tv.at[j])
        bi = tv[0].astype(jnp.int32)
        ci = tv[1].astype(jnp.int32)
        xv = tv[2]
        yv = tv[3]
        iota16 = lax.iota(jnp.int32, 16)
        tabs = (t0, t1, t2)
        ixrefs = (ix0, ix1, ix2)
        rdrefs = (rd0, rd1, rd2)
        gprefs = (gp0, gp1, gp2)
        xprefs = (xp0, xp1, xp2)
        gdrefs = (gd0, gd1, gd2)
        copies = []
        for s in range(3):
            H, W = SHAPES[s]
            stride = STRIDES[s]
            g0 = xv * jnp.float32(W / stride)
            g1 = yv * jnp.float32(H / stride)
            gi = g0.astype(jnp.int32)
            gj = g1.astype(jnp.int32)
            gm = GJMAX[s]
            ixrefs[s][...] = (bi * gm + gj) * gm + gi
            copies.append(pltpu.async_copy(
                tabs[s].at[ixrefs[s]], rdrefs[s], sem))
        for cp in copies:
            cp.wait()
        for s in range(3):
            rd = rdrefs[s]
            for c in range(4):
                outp[c] = plsc.load_gather(
                    rd, [iota16, jnp.full((16,), 64 + c, jnp.int32)])
            outp[4] = plsc.load_gather(rd, [iota16, 68 + ci])
            pltpu.sync_copy(outp.at[0:5], gprefs[s].at[wid])
            pltpu.sync_copy(outp.at[4], xprefs[s].at[pl.ds(g16, 16)])
            for c in range(64):
                outd[c] = plsc.load_gather(
                    rd, [iota16, jnp.full((16,), c, jnp.int32)])
            pltpu.sync_copy(outd, gdrefs[s].at[wid])


def _sc_gather(tt, t0, t1, t2):
    f32 = jnp.float32
    cp = pltpu.CompilerParams()
    fields = pltpu.CompilerParams.__dataclass_fields__
    if "needs_layout_passes" in fields:
        cp = dataclasses.replace(cp, needs_layout_passes=False)
    return pl.kernel(
        _sc_gather_body,
        compiler_params=cp,
        out_type=(
            jax.ShapeDtypeStruct((NG, 5, 16), f32),
            jax.ShapeDtypeStruct((NG, 5, 16), f32),
            jax.ShapeDtypeStruct((NG, 5, 16), f32),
            jax.ShapeDtypeStruct((N,), f32),
            jax.ShapeDtypeStruct((N,), f32),
            jax.ShapeDtypeStruct((N,), f32),
            jax.ShapeDtypeStruct((NG, 64, 16), f32),
            jax.ShapeDtypeStruct((NG, 64, 16), f32),
            jax.ShapeDtypeStruct((NG, 64, 16), f32),
        ),
        mesh=plsc.VectorSubcoreMesh(core_axis_name="c", subcore_axis_name="s"),
        scratch_types=[
            pltpu.VMEM((8, 16), f32),
            pltpu.VMEM((16,), jnp.int32),
            pltpu.VMEM((16,), jnp.int32),
            pltpu.VMEM((16,), jnp.int32),
            pltpu.VMEM((16, 256), f32),
            pltpu.VMEM((16, 256), f32),
            pltpu.VMEM((16, 256), f32),
            pltpu.VMEM((8, 16), f32),
            pltpu.VMEM((64, 16), f32),
            pltpu.SemaphoreType.DMA,
        ],
    )(tt, t0, t1, t2)


def kernel(pred0, pred1, pred2, dfl0, dfl1, dfl2, targets):
    p0t = jnp.transpose(pred0, (0, 2, 3, 1))
    p1t = jnp.transpose(pred1, (0, 2, 3, 1))
    p2t = jnp.transpose(pred2, (0, 2, 3, 1))
    d1t = jnp.transpose(dfl1, (0, 2, 3, 1))
    d2t = jnp.transpose(dfl2, (0, 2, 3, 1))
    tt = targets.T

    t0, t1, t2 = pl.pallas_call(
        _prep_body,
        grid=(1,),
        in_specs=[
            pl.BlockSpec((B, 8, 8, 84), lambda i: (0, 0, 0, 0)),
            pl.BlockSpec((B, 64, 8, 64), lambda i: (0, 0, 0, 0)),
            pl.BlockSpec((B, 2, 32, 84), lambda i: (0, 0, 0, 0)),
            pl.BlockSpec((B, 2, 32, 64), lambda i: (0, 0, 0, 0)),
            pl.BlockSpec((B, 1, 16, 84), lambda i: (0, 0, 0, 0)),
            pl.BlockSpec((B, 1, 16, 64), lambda i: (0, 0, 0, 0)),
        ],
        out_specs=[
            pl.BlockSpec((B * 64, 256), lambda i: (0, 0)),
            pl.BlockSpec((B * 4, 256), lambda i: (0, 0)),
            pl.BlockSpec((B, 256), lambda i: (0, 0)),
        ],
        out_shape=[
            jax.ShapeDtypeStruct((B * 64, 256), jnp.float32),
            jax.ShapeDtypeStruct((B * 4, 256), jnp.float32),
            jax.ShapeDtypeStruct((B, 256), jnp.float32),
        ],
    )(p0t, dfl0, p1t, d1t, p2t, d2t)

    gp0, gp1, gp2, xp0, xp1, xp2, gd0, gd1, gd2 = _sc_gather(tt, t0, t1, t2)
    tt3 = tt.reshape(6, NG, 16)

    ds = pl.pallas_call(
        _dense_body,
        grid=(B,),
        in_specs=[
            pl.BlockSpec((1, 64, 64, 84), lambda b: (b, 0, 0, 0)),
            pl.BlockSpec((1, 32, 32, 84), lambda b: (b, 0, 0, 0)),
            pl.BlockSpec((1, 16, 16, 84), lambda b: (b, 0, 0, 0)),
        ],
        out_specs=pl.BlockSpec((1, 8), lambda b: (0, 0)),
        out_shape=jax.ShapeDtypeStruct((1, 8), jnp.float32),
        scratch_shapes=[
            pltpu.VMEM((64, 64, 84), jnp.float32),
            pltpu.VMEM((32, 32, 84), jnp.float32),
            pltpu.VMEM((16, 16, 84), jnp.float32),
        ],
    )(p0t, p1t, p2t)

    out = pl.pallas_call(
        _combine_body,
        out_shape=jax.ShapeDtypeStruct((1, 4), jnp.float32),
    )(tt, tt3, gp0, gp1, gp2, xp0, xp1, xp2, gd0, gd1, gd2, ds)
    return out.reshape(4)
